# Initial kernel scaffold; baseline (speedup 1.0000x reference)
import jax
import jax.numpy as jnp
from jax.experimental import pallas as pl


def kernel(x, edge_index, batch, W1, att_src1, att_dst1, b1, W2, att_src2, att_dst2, b2, fc1_w, fc1_b, fc2_w, fc2_b):
    raise NotImplementedError("write your pallas kernel here")


# speedup vs baseline: 14.5434x; 14.5434x over previous
import dataclasses
import functools

import jax
import jax.numpy as jnp
from jax import lax
from jax.experimental import pallas as pl
from jax.experimental.pallas import tpu as pltpu
from jax.experimental.pallas import tpu_sc as plsc

N = 10000
E = 320000
F_IN = 128
HID = 128
NHID = 64
NG = 64
NC = 2
NS = 16
HHID = HID // NC
ESUB = E // NS
CH = 80
NCHK = ESUB // CH
ROWB = 1000
NBLK = N // ROWB

_SELU_L = 1.0507009873554805
_SELU_A = 1.6732632423543772


def _selu(x):
    return _SELU_L * jnp.where(x > 0, x, _SELU_A * (jnp.exp(x) - 1.0))


def _aux_cols(h, asr, adr):
    a_s = jnp.sum(h * asr, axis=1)
    a_d = jnp.sum(h * adr, axis=1)
    al = a_s + a_d
    base = jnp.maximum(al, 0.2 * al)
    return jnp.concatenate(
        [a_s[:, None], a_d[:, None], base[:, None],
         jnp.zeros((h.shape[0], 5), jnp.float32)], axis=1)


def _prep_body(x_ref, w_ref, asr_ref, adr_ref, h_ref, aux_ref):
    h = jnp.dot(x_ref[...], w_ref[...], preferred_element_type=jnp.float32,
                precision=lax.Precision.HIGHEST)
    h_ref[...] = h
    aux_ref[...] = _aux_cols(h, asr_ref[...], adr_ref[...])


def _prep(x, W, att_src, att_dst):
    return pl.pallas_call(
        _prep_body,
        grid=(NBLK,),
        in_specs=[
            pl.BlockSpec((ROWB, HID), lambda i: (i, 0)),
            pl.BlockSpec((HID, HID), lambda i: (0, 0)),
            pl.BlockSpec((1, HID), lambda i: (0, 0)),
            pl.BlockSpec((1, HID), lambda i: (0, 0)),
        ],
        out_specs=[
            pl.BlockSpec((ROWB, HID), lambda i: (i, 0)),
            pl.BlockSpec((ROWB, 8), lambda i: (i, 0)),
        ],
        out_shape=[
            jax.ShapeDtypeStruct((N, HID), jnp.float32),
            jax.ShapeDtypeStruct((N, 8), jnp.float32),
        ],
    )(x, W, att_src.reshape(1, HID), att_dst.reshape(1, HID))


def _sc_edge_body(h_hbm, aux_hbm, src_hbm, dst_hbm,
                  denomp_hbm, outp_hbm,
                  asrc_v, adst_v, base_v,
                  src_c, dst_c, ex_c, denom_v, rows_v, zbuf, out_sh, sem):
    core = lax.axis_index("c")
    sub = lax.axis_index("s")

    pltpu.sync_copy(aux_hbm.at[0, 0], asrc_v)
    pltpu.sync_copy(aux_hbm.at[1, 0], adst_v)
    pltpu.sync_copy(aux_hbm.at[2, 0], base_v)

    z16 = jnp.zeros((16,), jnp.float32)

    @pl.loop(0, 16)
    def _zb(i):
        for j in range(HHID // 16):
            zbuf[i, pl.ds(j * 16, 16)] = z16

    @pl.loop(0, N // 16)
    def _zd(k):
        denom_v[pl.ds(k * 16, 16)] = z16

    @pl.loop(0, (N // 16 + NS - 1) // NS)
    def _zo(k):
        idx = k * NS + sub

        @pl.when(idx < N // 16)
        def _():
            pltpu.sync_copy(zbuf, out_sh.at[pl.ds(idx * 16, 16)])

    plsc.subcore_barrier()

    @pl.loop(0, NCHK)
    def _edges(c):
        pltpu.sync_copy(src_hbm.at[sub, c], src_c.at[0])
        pltpu.sync_copy(dst_hbm.at[sub, c], dst_c.at[0])
        for k in range(CH // 16):
            s16 = src_c[0, pl.ds(k * 16, 16)]
            d16 = dst_c[0, pl.ds(k * 16, 16)]
            a_s = plsc.load_gather(asrc_v, [s16])
            a_d = plsc.load_gather(adst_v, [d16])
            b_d = plsc.load_gather(base_v, [d16])
            al = a_s + a_d
            al = jnp.maximum(al, 0.2 * al)
            ex = jnp.exp(al - b_d)
            ex_c[0, pl.ds(k * 16, 16)] = ex
            plsc.addupdate_scatter(denom_v, [d16], ex)
            src_c[0, pl.ds(k * 16, 16)] = s16 + core * N
        pltpu.async_copy(h_hbm.at[src_c.at[0]], rows_v, sem).wait()

        @pl.loop(0, CH)
        def _scale(i):
            cv = plsc.load_gather(ex_c.at[0], [jnp.full((16,), i, jnp.int32)])
            for j in range(HHID // 16):
                rows_v[i, pl.ds(j * 16, 16)] = rows_v[i, pl.ds(j * 16, 16)] * cv

        pltpu.sync_copy(rows_v, out_sh.at[dst_c.at[0]], add=True)

    @pl.when(core == 0)
    def _():
        pltpu.sync_copy(denom_v, denomp_hbm.at[sub, 0])

    plsc.subcore_barrier()

    @pl.loop(0, (N // 16 + NS - 1) // NS)
    def _wb(k):
        idx = k * NS + sub

        @pl.when(idx < N // 16)
        def _():
            pltpu.sync_copy(out_sh.at[pl.ds(idx * 16, 16)],
                            outp_hbm.at[core, pl.ds(idx * 16, 16)])


_SC_PARAMS = pltpu.CompilerParams()
for _f, _v in (("needs_layout_passes", False), ("use_tc_tiling_on_sc", False)):
    if _f in pltpu.CompilerParams.__dataclass_fields__:
        _SC_PARAMS = dataclasses.replace(_SC_PARAMS, **{_f: _v})


def _sc_edge(h_perm, aux, src2, dst2):
    mesh = plsc.VectorSubcoreMesh(core_axis_name="c", subcore_axis_name="s")
    fn = pl.kernel(
        _sc_edge_body,
        mesh=mesh,
        compiler_params=_SC_PARAMS,
        out_type=[
            jax.ShapeDtypeStruct((NS, 1, N), jnp.float32),
            jax.ShapeDtypeStruct((NC, N, HHID), jnp.float32),
        ],
        scratch_types=[
            pltpu.VMEM((N,), jnp.float32),
            pltpu.VMEM((N,), jnp.float32),
            pltpu.VMEM((N,), jnp.float32),
            pltpu.VMEM((1, CH), jnp.int32),
            pltpu.VMEM((1, CH), jnp.int32),
            pltpu.VMEM((1, CH), jnp.float32),
            pltpu.VMEM((N,), jnp.float32),
            pltpu.VMEM((CH, HHID), jnp.float32),
            pltpu.VMEM((16, HHID), jnp.float32),
            pltpu.VMEM_SHARED((N, HHID), jnp.float32),
            pltpu.SemaphoreType.DMA,
        ],
    )
    return fn(h_perm, aux, src2, dst2)


def _merge_body(outp_ref, denomp_ref, h_ref, b_ref, w_ref, asr_ref, adr_ref,
                h2_ref, aux_ref):
    acc = jnp.concatenate([outp_ref[0], outp_ref[1]], axis=1) + h_ref[...]
    den = jnp.sum(denomp_ref[...], axis=1) + (1.0 + 1e-16)
    node = _selu(acc / den[:, None] + b_ref[...])
    h2 = jnp.dot(node, w_ref[...], preferred_element_type=jnp.float32,
                 precision=lax.Precision.HIGHEST)
    h2_ref[...] = h2
    aux_ref[...] = _aux_cols(h2, asr_ref[...], adr_ref[...])


def _merge(outp, denomp, h, b, W, att_src, att_dst):
    return pl.pallas_call(
        _merge_body,
        grid=(NBLK,),
        in_specs=[
            pl.BlockSpec((NC, ROWB, HHID), lambda i: (0, i, 0)),
            pl.BlockSpec((ROWB, NS), lambda i: (i, 0)),
            pl.BlockSpec((ROWB, HID), lambda i: (i, 0)),
            pl.BlockSpec((1, HID), lambda i: (0, 0)),
            pl.BlockSpec((HID, HID), lambda i: (0, 0)),
            pl.BlockSpec((1, HID), lambda i: (0, 0)),
            pl.BlockSpec((1, HID), lambda i: (0, 0)),
        ],
        out_specs=[
            pl.BlockSpec((ROWB, HID), lambda i: (i, 0)),
            pl.BlockSpec((ROWB, 8), lambda i: (i, 0)),
        ],
        out_shape=[
            jax.ShapeDtypeStruct((N, HID), jnp.float32),
            jax.ShapeDtypeStruct((N, 8), jnp.float32),
        ],
    )(outp, denomp, h, b.reshape(1, HID), W,
      att_src.reshape(1, HID), att_dst.reshape(1, HID))


def _head_body(outp_ref, denomp_ref, h_ref, b_ref, batch_ref,
               fc1w_ref, fc1b_ref, fc2w_ref, fc2b_ref,
               out_ref, sums_ref, counts_ref):
    step = pl.program_id(0)

    @pl.when(step == 0)
    def _():
        sums_ref[...] = jnp.zeros_like(sums_ref)
        counts_ref[...] = jnp.zeros_like(counts_ref)

    acc = jnp.concatenate([outp_ref[0], outp_ref[1]], axis=1) + h_ref[...]
    den = jnp.sum(denomp_ref[...], axis=1) + (1.0 + 1e-16)
    node = _selu(acc / den[:, None] + b_ref[...])
    bt = batch_ref[0, 0, :]
    oh = (bt[:, None] == lax.broadcasted_iota(jnp.int32, (ROWB, NG), 1)
          ).astype(jnp.float32)
    sums_ref[...] += jnp.dot(oh.T, node, preferred_element_type=jnp.float32,
                             precision=lax.Precision.HIGHEST)
    counts_ref[...] += jnp.sum(oh, axis=0)[:, None]

    @pl.when(step == pl.num_programs(0) - 1)
    def _():
        pooled = sums_ref[...] / jnp.maximum(counts_ref[...], 1.0)
        g = _selu(pooled)
        g1 = _selu(jnp.dot(g, fc1w_ref[...], preferred_element_type=jnp.float32,
                           precision=lax.Precision.HIGHEST) + fc1b_ref[...])
        logits = jnp.dot(g1, fc2w_ref[...], preferred_element_type=jnp.float32,
                         precision=lax.Precision.HIGHEST) + fc2b_ref[...]
        col = lax.broadcasted_iota(jnp.int32, (NG, HID), 1)
        valid = col < 2
        m = jnp.max(jnp.where(valid, logits, -jnp.inf), axis=1, keepdims=True)
        s = jnp.sum(jnp.where(valid, jnp.exp(logits - m), 0.0),
                    axis=1, keepdims=True)
        out_ref[...] = logits - (m + jnp.log(s))


def _head(outp, denomp, h, b, batch3, fc1_w, fc1_b, fc2_wp, fc2_bp):
    return pl.pallas_call(
        _head_body,
        grid=(NBLK,),
        in_specs=[
            pl.BlockSpec((NC, ROWB, HHID), lambda i: (0, i, 0)),
            pl.BlockSpec((ROWB, NS), lambda i: (i, 0)),
            pl.BlockSpec((ROWB, HID), lambda i: (i, 0)),
            pl.BlockSpec((1, HID), lambda i: (0, 0)),
            pl.BlockSpec((1, 1, ROWB), lambda i: (i, 0, 0)),
            pl.BlockSpec((HID, NHID), lambda i: (0, 0)),
            pl.BlockSpec((1, NHID), lambda i: (0, 0)),
            pl.BlockSpec((NHID, HID), lambda i: (0, 0)),
            pl.BlockSpec((1, HID), lambda i: (0, 0)),
        ],
        out_specs=pl.BlockSpec((NG, HID), lambda i: (0, 0)),
        out_shape=jax.ShapeDtypeStruct((NG, HID), jnp.float32),
        scratch_shapes=[
            pltpu.VMEM((NG, HID), jnp.float32),
            pltpu.VMEM((NG, HID), jnp.float32),
        ],
    )(outp, denomp, h, b.reshape(1, HID), batch3,
      fc1_w, fc1_b.reshape(1, NHID), fc2_wp, fc2_bp)


def kernel(x, edge_index, batch, W1, att_src1, att_dst1, b1,
           W2, att_src2, att_dst2, b2, fc1_w, fc1_b, fc2_w, fc2_b):
    src2 = edge_index[0].reshape(NS, NCHK, CH)
    dst2 = edge_index[1].reshape(NS, NCHK, CH)
    batch3 = batch.reshape(NBLK, 1, ROWB)
    fc2_wp = jnp.pad(fc2_w, ((0, 0), (0, HID - 2)))
    fc2_bp = jnp.pad(fc2_b, (0, HID - 2)).reshape(1, HID)

    h1, aux1 = _prep(x, W1, att_src1, att_dst1)
    h1p = jnp.concatenate([h1[:, :HHID], h1[:, HHID:]], axis=0)
    denomp1, outp1 = _sc_edge(h1p, aux1.T[:3].reshape(3, 1, N), src2, dst2)
    h2, aux2 = _merge(outp1, denomp1.reshape(NS, N).T, h1, b1, W2,
                      att_src2, att_dst2)
    h2p = jnp.concatenate([h2[:, :HHID], h2[:, HHID:]], axis=0)
    denomp2, outp2 = _sc_edge(h2p, aux2.T[:3].reshape(3, 1, N), src2, dst2)
    out = _head(outp2, denomp2.reshape(NS, N).T, h2, b2, batch3,
                fc1_w, fc1_b, fc2_wp, fc2_bp)
    return out[:, :2]


# speedup vs baseline: 26.8367x; 1.8453x over previous
import dataclasses
import functools

import jax
import jax.numpy as jnp
from jax import lax
from jax.experimental import pallas as pl
from jax.experimental.pallas import tpu as pltpu
from jax.experimental.pallas import tpu_sc as plsc

N = 10000
E = 320000
F_IN = 128
HID = 128
NHID = 64
NG = 64
NC = 2
NS = 16
HHID = HID // NC
ESUB = E // NS
CH = 80
NCHK = ESUB // CH
SUP = 10
NSUP = NCHK // SUP
ROWB = 1000
NBLK = N // ROWB

_SELU_L = 1.0507009873554805
_SELU_A = 1.6732632423543772


def _selu(x):
    return _SELU_L * jnp.where(x > 0, x, _SELU_A * (jnp.exp(x) - 1.0))


def _aux_cols(h, asr, adr):
    a_s = jnp.sum(h * asr, axis=1)
    a_d = jnp.sum(h * adr, axis=1)
    al = a_s + a_d
    base = jnp.maximum(al, 0.2 * al)
    return jnp.concatenate(
        [a_s[:, None], a_d[:, None], base[:, None],
         jnp.zeros((h.shape[0], 5), jnp.float32)], axis=1)


def _prep_body(x_ref, w_ref, asr_ref, adr_ref, h_ref, aux_ref):
    h = jnp.dot(x_ref[...], w_ref[...], preferred_element_type=jnp.float32,
                precision=lax.Precision.HIGHEST)
    h_ref[...] = h
    aux_ref[...] = _aux_cols(h, asr_ref[...], adr_ref[...])


def _prep(x, W, att_src, att_dst):
    return pl.pallas_call(
        _prep_body,
        grid=(NBLK,),
        in_specs=[
            pl.BlockSpec((ROWB, HID), lambda i: (i, 0)),
            pl.BlockSpec((HID, HID), lambda i: (0, 0)),
            pl.BlockSpec((1, HID), lambda i: (0, 0)),
            pl.BlockSpec((1, HID), lambda i: (0, 0)),
        ],
        out_specs=[
            pl.BlockSpec((ROWB, HID), lambda i: (i, 0)),
            pl.BlockSpec((ROWB, 8), lambda i: (i, 0)),
        ],
        out_shape=[
            jax.ShapeDtypeStruct((N, HID), jnp.float32),
            jax.ShapeDtypeStruct((N, 8), jnp.float32),
        ],
    )(x, W, att_src.reshape(1, HID), att_dst.reshape(1, HID))


def _sc_edge_body(h_hbm, aux_hbm, src_hbm, dst_hbm,
                  denomp_hbm, outp_hbm,
                  asrc_v, adst_v, base_v,
                  src_s, dst_s, ex_s, denom_v,
                  rows_a, rows_b, zbuf, out_sh, sem_a, sem_b):
    core = lax.axis_index("c")
    sub = lax.axis_index("s")

    pltpu.sync_copy(aux_hbm.at[0, 0], asrc_v)
    pltpu.sync_copy(aux_hbm.at[1, 0], adst_v)
    pltpu.sync_copy(aux_hbm.at[2, 0], base_v)

    z16 = jnp.zeros((16,), jnp.float32)

    @pl.loop(0, 16)
    def _zb(i):
        for j in range(HHID // 16):
            zbuf[i, pl.ds(j * 16, 16)] = z16

    @pl.loop(0, N // 16)
    def _zd(k):
        denom_v[pl.ds(k * 16, 16)] = z16

    @pl.loop(0, (N // 16 + NS - 1) // NS)
    def _zo(k):
        idx = k * NS + sub

        @pl.when(idx < N // 16)
        def _():
            pltpu.sync_copy(zbuf, out_sh.at[pl.ds(idx * 16, 16)])

    plsc.subcore_barrier()

    def _scale_scatter(c, rows, gsem):
        pltpu.make_async_copy(h_hbm.at[src_s.at[c]], rows, gsem).wait()

        @pl.loop(0, CH)
        def _scale(i):
            cv = plsc.load_gather(ex_s, [jnp.full((16,), c * CH + i, jnp.int32)])
            for j in range(HHID // 16):
                rows[i, pl.ds(j * 16, 16)] = rows[i, pl.ds(j * 16, 16)] * cv

        pltpu.sync_copy(rows, out_sh.at[dst_s.at[c]], add=True)

    @pl.loop(0, NSUP)
    def _edges(S):
        pltpu.sync_copy(src_hbm.at[sub, pl.ds(S * SUP, SUP)], src_s)
        pltpu.sync_copy(dst_hbm.at[sub, pl.ds(S * SUP, SUP)], dst_s)

        @pl.loop(0, SUP)
        def _ex(c):
            for k in range(CH // 16):
                s16 = src_s[c, pl.ds(k * 16, 16)]
                d16 = dst_s[c, pl.ds(k * 16, 16)]
                a_s = plsc.load_gather(asrc_v, [s16])
                a_d = plsc.load_gather(adst_v, [d16])
                b_d = plsc.load_gather(base_v, [d16])
                al = a_s + a_d
                al = jnp.maximum(al, 0.2 * al)
                ex = jnp.exp(al - b_d)
                ex_s[pl.ds(c * CH + k * 16, 16)] = ex
                plsc.addupdate_scatter(denom_v, [d16], ex)
                src_s[c, pl.ds(k * 16, 16)] = s16 + core * N

        pltpu.async_copy(h_hbm.at[src_s.at[0]], rows_a, sem_a)

        @pl.loop(0, SUP // 2)
        def _pipe(t):
            pltpu.async_copy(h_hbm.at[src_s.at[2 * t + 1]], rows_b, sem_b)
            _scale_scatter(2 * t, rows_a, sem_a)

            @pl.when(2 * t + 2 < SUP)
            def _():
                pltpu.async_copy(h_hbm.at[src_s.at[2 * t + 2]], rows_a, sem_a)

            _scale_scatter(2 * t + 1, rows_b, sem_b)

    @pl.when(core == 0)
    def _():
        pltpu.sync_copy(denom_v, denomp_hbm.at[sub, 0])

    plsc.subcore_barrier()

    @pl.loop(0, (N // 16 + NS - 1) // NS)
    def _wb(k):
        idx = k * NS + sub

        @pl.when(idx < N // 16)
        def _():
            pltpu.sync_copy(out_sh.at[pl.ds(idx * 16, 16)],
                            outp_hbm.at[core, pl.ds(idx * 16, 16)])


_SC_PARAMS = pltpu.CompilerParams()
for _f, _v in (("needs_layout_passes", False), ("use_tc_tiling_on_sc", False)):
    if _f in pltpu.CompilerParams.__dataclass_fields__:
        _SC_PARAMS = dataclasses.replace(_SC_PARAMS, **{_f: _v})


def _sc_edge(h_perm, aux, src2, dst2):
    mesh = plsc.VectorSubcoreMesh(core_axis_name="c", subcore_axis_name="s")
    fn = pl.kernel(
        _sc_edge_body,
        mesh=mesh,
        compiler_params=_SC_PARAMS,
        out_type=[
            jax.ShapeDtypeStruct((NS, 1, N), jnp.float32),
            jax.ShapeDtypeStruct((NC, N, HHID), jnp.float32),
        ],
        scratch_types=[
            pltpu.VMEM((N,), jnp.float32),
            pltpu.VMEM((N,), jnp.float32),
            pltpu.VMEM((N,), jnp.float32),
            pltpu.VMEM((SUP, CH), jnp.int32),
            pltpu.VMEM((SUP, CH), jnp.int32),
            pltpu.VMEM((SUP * CH,), jnp.float32),
            pltpu.VMEM((N,), jnp.float32),
            pltpu.VMEM((CH, HHID), jnp.float32),
            pltpu.VMEM((CH, HHID), jnp.float32),
            pltpu.VMEM((16, HHID), jnp.float32),
            pltpu.VMEM_SHARED((N, HHID), jnp.float32),
            pltpu.SemaphoreType.DMA,
            pltpu.SemaphoreType.DMA,
        ],
    )
    return fn(h_perm, aux, src2, dst2)


def _merge_body(outp_ref, denomp_ref, h_ref, b_ref, w_ref, asr_ref, adr_ref,
                h2_ref, aux_ref):
    acc = jnp.concatenate([outp_ref[0], outp_ref[1]], axis=1) + h_ref[...]
    den = jnp.sum(denomp_ref[...], axis=1) + (1.0 + 1e-16)
    node = _selu(acc / den[:, None] + b_ref[...])
    h2 = jnp.dot(node, w_ref[...], preferred_element_type=jnp.float32,
                 precision=lax.Precision.HIGHEST)
    h2_ref[...] = h2
    aux_ref[...] = _aux_cols(h2, asr_ref[...], adr_ref[...])


def _merge(outp, denomp, h, b, W, att_src, att_dst):
    return pl.pallas_call(
        _merge_body,
        grid=(NBLK,),
        in_specs=[
            pl.BlockSpec((NC, ROWB, HHID), lambda i: (0, i, 0)),
            pl.BlockSpec((ROWB, NS), lambda i: (i, 0)),
            pl.BlockSpec((ROWB, HID), lambda i: (i, 0)),
            pl.BlockSpec((1, HID), lambda i: (0, 0)),
            pl.BlockSpec((HID, HID), lambda i: (0, 0)),
            pl.BlockSpec((1, HID), lambda i: (0, 0)),
            pl.BlockSpec((1, HID), lambda i: (0, 0)),
        ],
        out_specs=[
            pl.BlockSpec((ROWB, HID), lambda i: (i, 0)),
            pl.BlockSpec((ROWB, 8), lambda i: (i, 0)),
        ],
        out_shape=[
            jax.ShapeDtypeStruct((N, HID), jnp.float32),
            jax.ShapeDtypeStruct((N, 8), jnp.float32),
        ],
    )(outp, denomp, h, b.reshape(1, HID), W,
      att_src.reshape(1, HID), att_dst.reshape(1, HID))


def _head_body(outp_ref, denomp_ref, h_ref, b_ref, batch_ref,
               fc1w_ref, fc1b_ref, fc2w_ref, fc2b_ref,
               out_ref, sums_ref, counts_ref):
    step = pl.program_id(0)

    @pl.when(step == 0)
    def _():
        sums_ref[...] = jnp.zeros_like(sums_ref)
        counts_ref[...] = jnp.zeros_like(counts_ref)

    acc = jnp.concatenate([outp_ref[0], outp_ref[1]], axis=1) + h_ref[...]
    den = jnp.sum(denomp_ref[...], axis=1) + (1.0 + 1e-16)
    node = _selu(acc / den[:, None] + b_ref[...])
    bt = batch_ref[0, 0, :]
    oh = (bt[:, None] == lax.broadcasted_iota(jnp.int32, (ROWB, NG), 1)
          ).astype(jnp.float32)
    sums_ref[...] += jnp.dot(oh.T, node, preferred_element_type=jnp.float32,
                             precision=lax.Precision.HIGHEST)
    counts_ref[...] += jnp.sum(oh, axis=0)[:, None]

    @pl.when(step == pl.num_programs(0) - 1)
    def _():
        pooled = sums_ref[...] / jnp.maximum(counts_ref[...], 1.0)
        g = _selu(pooled)
        g1 = _selu(jnp.dot(g, fc1w_ref[...], preferred_element_type=jnp.float32,
                           precision=lax.Precision.HIGHEST) + fc1b_ref[...])
        logits = jnp.dot(g1, fc2w_ref[...], preferred_element_type=jnp.float32,
                         precision=lax.Precision.HIGHEST) + fc2b_ref[...]
        col = lax.broadcasted_iota(jnp.int32, (NG, HID), 1)
        valid = col < 2
        m = jnp.max(jnp.where(valid, logits, -jnp.inf), axis=1, keepdims=True)
        s = jnp.sum(jnp.where(valid, jnp.exp(logits - m), 0.0),
                    axis=1, keepdims=True)
        out_ref[...] = logits - (m + jnp.log(s))


def _head(outp, denomp, h, b, batch3, fc1_w, fc1_b, fc2_wp, fc2_bp):
    return pl.pallas_call(
        _head_body,
        grid=(NBLK,),
        in_specs=[
            pl.BlockSpec((NC, ROWB, HHID), lambda i: (0, i, 0)),
            pl.BlockSpec((ROWB, NS), lambda i: (i, 0)),
            pl.BlockSpec((ROWB, HID), lambda i: (i, 0)),
            pl.BlockSpec((1, HID), lambda i: (0, 0)),
            pl.BlockSpec((1, 1, ROWB), lambda i: (i, 0, 0)),
            pl.BlockSpec((HID, NHID), lambda i: (0, 0)),
            pl.BlockSpec((1, NHID), lambda i: (0, 0)),
            pl.BlockSpec((NHID, HID), lambda i: (0, 0)),
            pl.BlockSpec((1, HID), lambda i: (0, 0)),
        ],
        out_specs=pl.BlockSpec((NG, HID), lambda i: (0, 0)),
        out_shape=jax.ShapeDtypeStruct((NG, HID), jnp.float32),
        scratch_shapes=[
            pltpu.VMEM((NG, HID), jnp.float32),
            pltpu.VMEM((NG, HID), jnp.float32),
        ],
    )(outp, denomp, h, b.reshape(1, HID), batch3,
      fc1_w, fc1_b.reshape(1, NHID), fc2_wp, fc2_bp)


def kernel(x, edge_index, batch, W1, att_src1, att_dst1, b1,
           W2, att_src2, att_dst2, b2, fc1_w, fc1_b, fc2_w, fc2_b):
    src2 = edge_index[0].reshape(NS, NCHK, CH)
    dst2 = edge_index[1].reshape(NS, NCHK, CH)
    batch3 = batch.reshape(NBLK, 1, ROWB)
    fc2_wp = jnp.pad(fc2_w, ((0, 0), (0, HID - 2)))
    fc2_bp = jnp.pad(fc2_b, (0, HID - 2)).reshape(1, HID)

    h1, aux1 = _prep(x, W1, att_src1, att_dst1)
    h1p = jnp.concatenate([h1[:, :HHID], h1[:, HHID:]], axis=0)
    denomp1, outp1 = _sc_edge(h1p, aux1.T[:3].reshape(3, 1, N), src2, dst2)
    h2, aux2 = _merge(outp1, denomp1.reshape(NS, N).T, h1, b1, W2,
                      att_src2, att_dst2)
    h2p = jnp.concatenate([h2[:, :HHID], h2[:, HHID:]], axis=0)
    denomp2, outp2 = _sc_edge(h2p, aux2.T[:3].reshape(3, 1, N), src2, dst2)
    out = _head(outp2, denomp2.reshape(NS, N).T, h2, b2, batch3,
                fc1_w, fc1_b, fc2_wp, fc2_bp)
    return out[:, :2]


# speedup vs baseline: 30.4153x; 1.1333x over previous
import dataclasses
import functools

import jax
import jax.numpy as jnp
from jax import lax
from jax.experimental import pallas as pl
from jax.experimental.pallas import tpu as pltpu
from jax.experimental.pallas import tpu_sc as plsc

N = 10000
E = 320000
F_IN = 128
HID = 128
NHID = 64
NG = 64
NC = 2
NS = 16
HHID = HID // NC
ESUB = E // NS
CH = 80
NCHK = ESUB // CH
SUP = 10
NSUP = NCHK // SUP
ROWB = 1000
NBLK = N // ROWB

_SELU_L = 1.0507009873554805
_SELU_A = 1.6732632423543772


def _selu(x):
    return _SELU_L * jnp.where(x > 0, x, _SELU_A * (jnp.exp(x) - 1.0))


def _aux_cols(h, asr, adr):
    a_s = jnp.sum(h * asr, axis=1)
    a_d = jnp.sum(h * adr, axis=1)
    al = a_s + a_d
    base = jnp.maximum(al, 0.2 * al)
    return jnp.concatenate(
        [a_s[:, None], a_d[:, None], base[:, None],
         jnp.zeros((h.shape[0], 5), jnp.float32)], axis=1)


def _prep_body(x_ref, w_ref, asr_ref, adr_ref, h_ref, aux_ref):
    h = jnp.dot(x_ref[...], w_ref[...], preferred_element_type=jnp.float32,
                precision=lax.Precision.HIGHEST)
    h_ref[...] = h
    aux_ref[...] = _aux_cols(h, asr_ref[...], adr_ref[...])


def _prep(x, W, att_src, att_dst):
    return pl.pallas_call(
        _prep_body,
        grid=(NBLK,),
        in_specs=[
            pl.BlockSpec((ROWB, HID), lambda i: (i, 0)),
            pl.BlockSpec((HID, HID), lambda i: (0, 0)),
            pl.BlockSpec((1, HID), lambda i: (0, 0)),
            pl.BlockSpec((1, HID), lambda i: (0, 0)),
        ],
        out_specs=[
            pl.BlockSpec((ROWB, HID), lambda i: (i, 0)),
            pl.BlockSpec((ROWB, 8), lambda i: (i, 0)),
        ],
        out_shape=[
            jax.ShapeDtypeStruct((N, HID), jnp.float32),
            jax.ShapeDtypeStruct((N, 8), jnp.float32),
        ],
    )(x, W, att_src.reshape(1, HID), att_dst.reshape(1, HID))


def _sc_edge_body(h_hbm, aux_hbm, src_hbm, dst_hbm,
                  denomp_hbm, outp_hbm,
                  asrc_v, adst_v, base_v,
                  src_s, dst_s, ex_s, denom_v,
                  rows_a, rows_b, rows_c, rows_d, zbuf, out_sh,
                  gsem_a, gsem_b, gsem_c, gsem_d,
                  ssem_a, ssem_b, ssem_c, ssem_d):
    core = lax.axis_index("c")
    sub = lax.axis_index("s")

    pltpu.sync_copy(aux_hbm.at[0, 0], asrc_v)
    pltpu.sync_copy(aux_hbm.at[1, 0], adst_v)
    pltpu.sync_copy(aux_hbm.at[2, 0], base_v)

    z16 = jnp.zeros((16,), jnp.float32)

    @pl.loop(0, 16)
    def _zb(i):
        for j in range(HHID // 16):
            zbuf[i, pl.ds(j * 16, 16)] = z16

    @pl.loop(0, N // 16)
    def _zd(k):
        denom_v[pl.ds(k * 16, 16)] = z16

    @pl.loop(0, (N // 16 + NS - 1) // NS)
    def _zo(k):
        idx = k * NS + sub

        @pl.when(idx < N // 16)
        def _():
            pltpu.sync_copy(zbuf, out_sh.at[pl.ds(idx * 16, 16)])

    plsc.subcore_barrier()

    rr = (rows_a, rows_b, rows_c, rows_d)
    gs = (gsem_a, gsem_b, gsem_c, gsem_d)
    ss = (ssem_a, ssem_b, ssem_c, ssem_d)

    def _wait_bytes(rows, sem):
        pltpu.make_async_copy(h_hbm.at[src_s.at[0]], rows, sem).wait()

    def _scale(c, rows):
        @pl.loop(0, CH)
        def _s(i):
            cv = plsc.load_gather(ex_s, [jnp.full((16,), c * CH + i, jnp.int32)])
            for j in range(HHID // 16):
                rows[i, pl.ds(j * 16, 16)] = rows[i, pl.ds(j * 16, 16)] * cv

    @pl.loop(0, NSUP)
    def _edges(S):
        pltpu.sync_copy(src_hbm.at[sub, pl.ds(S * SUP, SUP)], src_s)
        pltpu.sync_copy(dst_hbm.at[sub, pl.ds(S * SUP, SUP)], dst_s)

        @pl.loop(0, SUP)
        def _ex(c):
            for k in range(CH // 16):
                s16 = src_s[c, pl.ds(k * 16, 16)]
                d16 = dst_s[c, pl.ds(k * 16, 16)]
                a_s = plsc.load_gather(asrc_v, [s16])
                a_d = plsc.load_gather(adst_v, [d16])
                b_d = plsc.load_gather(base_v, [d16])
                al = a_s + a_d
                al = jnp.maximum(al, 0.2 * al)
                ex = jnp.exp(al - b_d)
                ex_s[pl.ds(c * CH + k * 16, 16)] = ex
                plsc.addupdate_scatter(denom_v, [d16], ex)
                src_s[c, pl.ds(k * 16, 16)] = s16 + core * N

        for c in range(3):
            pltpu.async_copy(h_hbm.at[src_s.at[c]], rr[c], gs[c])
        for c in range(SUP):
            b = c % 4
            _wait_bytes(rr[b], gs[b])
            _scale(c, rr[b])
            pltpu.async_copy(rr[b], out_sh.at[dst_s.at[c]], ss[b], add=True)
            if c + 3 < SUP:
                b3 = (c + 3) % 4
                if c - 1 >= 0:
                    _wait_bytes(rr[b3], ss[b3])
                pltpu.async_copy(h_hbm.at[src_s.at[c + 3]], rr[b3], gs[b3])
        for c in range(SUP - 4, SUP):
            _wait_bytes(rr[c % 4], ss[c % 4])

    @pl.when(core == 0)
    def _():
        pltpu.sync_copy(denom_v, denomp_hbm.at[sub, 0])

    plsc.subcore_barrier()

    @pl.loop(0, (N // 16 + NS - 1) // NS)
    def _wb(k):
        idx = k * NS + sub

        @pl.when(idx < N // 16)
        def _():
            pltpu.sync_copy(out_sh.at[pl.ds(idx * 16, 16)],
                            outp_hbm.at[core, pl.ds(idx * 16, 16)])


_SC_PARAMS = pltpu.CompilerParams()
for _f, _v in (("needs_layout_passes", False), ("use_tc_tiling_on_sc", False)):
    if _f in pltpu.CompilerParams.__dataclass_fields__:
        _SC_PARAMS = dataclasses.replace(_SC_PARAMS, **{_f: _v})


def _sc_edge(h_perm, aux, src2, dst2):
    mesh = plsc.VectorSubcoreMesh(core_axis_name="c", subcore_axis_name="s")
    fn = pl.kernel(
        _sc_edge_body,
        mesh=mesh,
        compiler_params=_SC_PARAMS,
        out_type=[
            jax.ShapeDtypeStruct((NS, 1, N), jnp.float32),
            jax.ShapeDtypeStruct((NC, N, HHID), jnp.float32),
        ],
        scratch_types=[
            pltpu.VMEM((N,), jnp.float32),
            pltpu.VMEM((N,), jnp.float32),
            pltpu.VMEM((N,), jnp.float32),
            pltpu.VMEM((SUP, CH), jnp.int32),
            pltpu.VMEM((SUP, CH), jnp.int32),
            pltpu.VMEM((SUP * CH,), jnp.float32),
            pltpu.VMEM((N,), jnp.float32),
            pltpu.VMEM((CH, HHID), jnp.float32),
            pltpu.VMEM((CH, HHID), jnp.float32),
            pltpu.VMEM((CH, HHID), jnp.float32),
            pltpu.VMEM((CH, HHID), jnp.float32),
            pltpu.VMEM((16, HHID), jnp.float32),
            pltpu.VMEM_SHARED((N, HHID), jnp.float32),
        ] + [pltpu.SemaphoreType.DMA] * 8,
    )
    return fn(h_perm, aux, src2, dst2)


def _merge_body(outp_ref, denomp_ref, h_ref, b_ref, w_ref, asr_ref, adr_ref,
                h2_ref, aux_ref):
    acc = jnp.concatenate([outp_ref[0], outp_ref[1]], axis=1) + h_ref[...]
    den = jnp.sum(denomp_ref[...], axis=1) + (1.0 + 1e-16)
    node = _selu(acc / den[:, None] + b_ref[...])
    h2 = jnp.dot(node, w_ref[...], preferred_element_type=jnp.float32,
                 precision=lax.Precision.HIGHEST)
    h2_ref[...] = h2
    aux_ref[...] = _aux_cols(h2, asr_ref[...], adr_ref[...])


def _merge(outp, denomp, h, b, W, att_src, att_dst):
    return pl.pallas_call(
        _merge_body,
        grid=(NBLK,),
        in_specs=[
            pl.BlockSpec((NC, ROWB, HHID), lambda i: (0, i, 0)),
            pl.BlockSpec((ROWB, NS), lambda i: (i, 0)),
            pl.BlockSpec((ROWB, HID), lambda i: (i, 0)),
            pl.BlockSpec((1, HID), lambda i: (0, 0)),
            pl.BlockSpec((HID, HID), lambda i: (0, 0)),
            pl.BlockSpec((1, HID), lambda i: (0, 0)),
            pl.BlockSpec((1, HID), lambda i: (0, 0)),
        ],
        out_specs=[
            pl.BlockSpec((ROWB, HID), lambda i: (i, 0)),
            pl.BlockSpec((ROWB, 8), lambda i: (i, 0)),
        ],
        out_shape=[
            jax.ShapeDtypeStruct((N, HID), jnp.float32),
            jax.ShapeDtypeStruct((N, 8), jnp.float32),
        ],
    )(outp, denomp, h, b.reshape(1, HID), W,
      att_src.reshape(1, HID), att_dst.reshape(1, HID))


def _head_body(outp_ref, denomp_ref, h_ref, b_ref, batch_ref,
               fc1w_ref, fc1b_ref, fc2w_ref, fc2b_ref,
               out_ref, sums_ref, counts_ref):
    step = pl.program_id(0)

    @pl.when(step == 0)
    def _():
        sums_ref[...] = jnp.zeros_like(sums_ref)
        counts_ref[...] = jnp.zeros_like(counts_ref)

    acc = jnp.concatenate([outp_ref[0], outp_ref[1]], axis=1) + h_ref[...]
    den = jnp.sum(denomp_ref[...], axis=1) + (1.0 + 1e-16)
    node = _selu(acc / den[:, None] + b_ref[...])
    bt = batch_ref[0, 0, :]
    oh = (bt[:, None] == lax.broadcasted_iota(jnp.int32, (ROWB, NG), 1)
          ).astype(jnp.float32)
    sums_ref[...] += jnp.dot(oh.T, node, preferred_element_type=jnp.float32,
                             precision=lax.Precision.HIGHEST)
    counts_ref[...] += jnp.sum(oh, axis=0)[:, None]

    @pl.when(step == pl.num_programs(0) - 1)
    def _():
        pooled = sums_ref[...] / jnp.maximum(counts_ref[...], 1.0)
        g = _selu(pooled)
        g1 = _selu(jnp.dot(g, fc1w_ref[...], preferred_element_type=jnp.float32,
                           precision=lax.Precision.HIGHEST) + fc1b_ref[...])
        logits = jnp.dot(g1, fc2w_ref[...], preferred_element_type=jnp.float32,
                         precision=lax.Precision.HIGHEST) + fc2b_ref[...]
        col = lax.broadcasted_iota(jnp.int32, (NG, HID), 1)
        valid = col < 2
        m = jnp.max(jnp.where(valid, logits, -jnp.inf), axis=1, keepdims=True)
        s = jnp.sum(jnp.where(valid, jnp.exp(logits - m), 0.0),
                    axis=1, keepdims=True)
        out_ref[...] = logits - (m + jnp.log(s))


def _head(outp, denomp, h, b, batch3, fc1_w, fc1_b, fc2_wp, fc2_bp):
    return pl.pallas_call(
        _head_body,
        grid=(NBLK,),
        in_specs=[
            pl.BlockSpec((NC, ROWB, HHID), lambda i: (0, i, 0)),
            pl.BlockSpec((ROWB, NS), lambda i: (i, 0)),
            pl.BlockSpec((ROWB, HID), lambda i: (i, 0)),
            pl.BlockSpec((1, HID), lambda i: (0, 0)),
            pl.BlockSpec((1, 1, ROWB), lambda i: (i, 0, 0)),
            pl.BlockSpec((HID, NHID), lambda i: (0, 0)),
            pl.BlockSpec((1, NHID), lambda i: (0, 0)),
            pl.BlockSpec((NHID, HID), lambda i: (0, 0)),
            pl.BlockSpec((1, HID), lambda i: (0, 0)),
        ],
        out_specs=pl.BlockSpec((NG, HID), lambda i: (0, 0)),
        out_shape=jax.ShapeDtypeStruct((NG, HID), jnp.float32),
        scratch_shapes=[
            pltpu.VMEM((NG, HID), jnp.float32),
            pltpu.VMEM((NG, HID), jnp.float32),
        ],
    )(outp, denomp, h, b.reshape(1, HID), batch3,
      fc1_w, fc1_b.reshape(1, NHID), fc2_wp, fc2_bp)


def kernel(x, edge_index, batch, W1, att_src1, att_dst1, b1,
           W2, att_src2, att_dst2, b2, fc1_w, fc1_b, fc2_w, fc2_b):
    src2 = edge_index[0].reshape(NS, NCHK, CH)
    dst2 = edge_index[1].reshape(NS, NCHK, CH)
    batch3 = batch.reshape(NBLK, 1, ROWB)
    fc2_wp = jnp.pad(fc2_w, ((0, 0), (0, HID - 2)))
    fc2_bp = jnp.pad(fc2_b, (0, HID - 2)).reshape(1, HID)

    h1, aux1 = _prep(x, W1, att_src1, att_dst1)
    h1p = jnp.concatenate([h1[:, :HHID], h1[:, HHID:]], axis=0)
    denomp1, outp1 = _sc_edge(h1p, aux1.T[:3].reshape(3, 1, N), src2, dst2)
    h2, aux2 = _merge(outp1, denomp1.reshape(NS, N).T, h1, b1, W2,
                      att_src2, att_dst2)
    h2p = jnp.concatenate([h2[:, :HHID], h2[:, HHID:]], axis=0)
    denomp2, outp2 = _sc_edge(h2p, aux2.T[:3].reshape(3, 1, N), src2, dst2)
    out = _head(outp2, denomp2.reshape(NS, N).T, h2, b2, batch3,
                fc1_w, fc1_b, fc2_wp, fc2_bp)
    return out[:, :2]


# speedup vs baseline: 37.2936x; 1.2261x over previous
import dataclasses
import functools

import jax
import jax.numpy as jnp
from jax import lax
from jax.experimental import pallas as pl
from jax.experimental.pallas import tpu as pltpu
from jax.experimental.pallas import tpu_sc as plsc

N = 10000
E = 320000
F_IN = 128
HID = 128
NHID = 64
NG = 64
NC = 2
NS = 16
HHID = HID // NC
ESUB = E // NS
CH = 80
NCHK = ESUB // CH
SUP = 10
NSUP = NCHK // SUP
ROWB = 1000
NBLK = N // ROWB

_SELU_L = 1.0507009873554805
_SELU_A = 1.6732632423543772


def _selu(x):
    return _SELU_L * jnp.where(x > 0, x, _SELU_A * (jnp.exp(x) - 1.0))


def _write_aux(aux_ref, h, asr, adr):
    a_s = jnp.sum(h * asr, axis=1)
    a_d = jnp.sum(h * adr, axis=1)
    al = a_s + a_d
    base = jnp.maximum(al, 0.2 * al)
    aux_ref[0, 0, :] = a_s
    aux_ref[0, 1, :] = a_d
    aux_ref[0, 2, :] = base


def _prep_body(x_ref, w_ref, asr_ref, adr_ref, h_ref, hp_ref, aux_ref):
    h = jnp.dot(x_ref[...], w_ref[...], preferred_element_type=jnp.float32,
                precision=lax.Precision.HIGHEST)
    h_ref[...] = h
    hp_ref[0] = h[:, :HHID]
    hp_ref[1] = h[:, HHID:]
    _write_aux(aux_ref, h, asr_ref[...], adr_ref[...])


def _prep(x, W, att_src, att_dst):
    return pl.pallas_call(
        _prep_body,
        grid=(NBLK,),
        in_specs=[
            pl.BlockSpec((ROWB, HID), lambda i: (i, 0)),
            pl.BlockSpec((HID, HID), lambda i: (0, 0)),
            pl.BlockSpec((1, HID), lambda i: (0, 0)),
            pl.BlockSpec((1, HID), lambda i: (0, 0)),
        ],
        out_specs=[
            pl.BlockSpec((ROWB, HID), lambda i: (i, 0)),
            pl.BlockSpec((NC, ROWB, HHID), lambda i: (0, i, 0)),
            pl.BlockSpec((1, 8, ROWB), lambda i: (i, 0, 0)),
        ],
        out_shape=[
            jax.ShapeDtypeStruct((N, HID), jnp.float32),
            jax.ShapeDtypeStruct((NC, N, HHID), jnp.float32),
            jax.ShapeDtypeStruct((NBLK, 8, ROWB), jnp.float32),
        ],
    )(x, W, att_src.reshape(1, HID), att_dst.reshape(1, HID))


def _sc_edge_body(h_hbm, aux_hbm, src_hbm, dst_hbm,
                  denomp_hbm, outp_hbm,
                  asrc_v, adst_v, base_v,
                  src_s, dst_s, ex_s, denom_v,
                  rows_a, rows_b, rows_c, rows_d, zbuf, out_sh,
                  gsem_a, gsem_b, gsem_c, gsem_d,
                  ssem_a, ssem_b, ssem_c, ssem_d):
    core = lax.axis_index("c")
    sub = lax.axis_index("s")

    for b in range(NBLK):
        for r, tab in ((0, asrc_v), (1, adst_v), (2, base_v)):
            pltpu.async_copy(aux_hbm.at[b, r], tab.at[pl.ds(b * ROWB, ROWB)],
                             gsem_a)
    for b in range(NBLK):
        for r, tab in ((0, asrc_v), (1, adst_v), (2, base_v)):
            pltpu.make_async_copy(aux_hbm.at[b, r],
                                  tab.at[pl.ds(b * ROWB, ROWB)], gsem_a).wait()

    z16 = jnp.zeros((16,), jnp.float32)

    @pl.loop(0, 16)
    def _zb(i):
        for j in range(HHID // 16):
            zbuf[i, pl.ds(j * 16, 16)] = z16

    @pl.loop(0, N // 16)
    def _zd(k):
        denom_v[pl.ds(k * 16, 16)] = z16

    @pl.loop(0, (N // 16 + NS - 1) // NS)
    def _zo(k):
        idx = k * NS + sub

        @pl.when(idx < N // 16)
        def _():
            pltpu.sync_copy(zbuf, out_sh.at[pl.ds(idx * 16, 16)])

    plsc.subcore_barrier()

    rr = (rows_a, rows_b, rows_c, rows_d)
    gs = (gsem_a, gsem_b, gsem_c, gsem_d)
    ss = (ssem_a, ssem_b, ssem_c, ssem_d)

    def _wait_bytes(rows, sem):
        pltpu.make_async_copy(h_hbm.at[src_s.at[0]], rows, sem).wait()

    def _scale(c, rows):
        @plsc.parallel_loop(0, CH, unroll=4)
        def _s(i):
            cv = plsc.load_gather(ex_s, [jnp.full((16,), c * CH + i, jnp.int32)])
            for j in range(HHID // 16):
                rows[i, pl.ds(j * 16, 16)] = rows[i, pl.ds(j * 16, 16)] * cv

    @pl.loop(0, NSUP)
    def _edges(S):
        pltpu.sync_copy(src_hbm.at[sub, pl.ds(S * SUP, SUP)], src_s)
        pltpu.sync_copy(dst_hbm.at[sub, pl.ds(S * SUP, SUP)], dst_s)

        @pl.loop(0, SUP)
        def _ex(c):
            for k in range(CH // 16):
                s16 = src_s[c, pl.ds(k * 16, 16)]
                d16 = dst_s[c, pl.ds(k * 16, 16)]
                a_s = plsc.load_gather(asrc_v, [s16])
                a_d = plsc.load_gather(adst_v, [d16])
                b_d = plsc.load_gather(base_v, [d16])
                al = a_s + a_d
                al = jnp.maximum(al, 0.2 * al)
                ex = jnp.exp(al - b_d)
                ex_s[pl.ds(c * CH + k * 16, 16)] = ex
                plsc.addupdate_scatter(denom_v, [d16], ex)
                src_s[c, pl.ds(k * 16, 16)] = s16 + core * N

        for c in range(3):
            pltpu.async_copy(h_hbm.at[src_s.at[c]], rr[c], gs[c])
        for c in range(SUP):
            b = c % 4
            _wait_bytes(rr[b], gs[b])
            _scale(c, rr[b])
            pltpu.async_copy(rr[b], out_sh.at[dst_s.at[c]], ss[b], add=True)
            if c + 3 < SUP:
                b3 = (c + 3) % 4
                if c - 1 >= 0:
                    _wait_bytes(rr[b3], ss[b3])
                pltpu.async_copy(h_hbm.at[src_s.at[c + 3]], rr[b3], gs[b3])
        for c in range(SUP - 4, SUP):
            _wait_bytes(rr[c % 4], ss[c % 4])

    @pl.when(core == 0)
    def _():
        for b in range(NBLK):
            pltpu.async_copy(denom_v.at[pl.ds(b * ROWB, ROWB)],
                             denomp_hbm.at[b, sub], gsem_a)
        for b in range(NBLK):
            pltpu.make_async_copy(denom_v.at[pl.ds(b * ROWB, ROWB)],
                                  denomp_hbm.at[b, sub], gsem_a).wait()

    plsc.subcore_barrier()

    @pl.loop(0, (N // 16 + NS - 1) // NS)
    def _wb(k):
        idx = k * NS + sub

        @pl.when(idx < N // 16)
        def _():
            pltpu.sync_copy(out_sh.at[pl.ds(idx * 16, 16)],
                            outp_hbm.at[core, pl.ds(idx * 16, 16)])


_SC_PARAMS = pltpu.CompilerParams()
for _f, _v in (("needs_layout_passes", False), ("use_tc_tiling_on_sc", False)):
    if _f in pltpu.CompilerParams.__dataclass_fields__:
        _SC_PARAMS = dataclasses.replace(_SC_PARAMS, **{_f: _v})


def _sc_edge(h_perm, aux, src2, dst2):
    mesh = plsc.VectorSubcoreMesh(core_axis_name="c", subcore_axis_name="s")
    fn = pl.kernel(
        _sc_edge_body,
        mesh=mesh,
        compiler_params=_SC_PARAMS,
        out_type=[
            jax.ShapeDtypeStruct((NBLK, NS, ROWB), jnp.float32),
            jax.ShapeDtypeStruct((NC, N, HHID), jnp.float32),
        ],
        scratch_types=[
            pltpu.VMEM((N,), jnp.float32),
            pltpu.VMEM((N,), jnp.float32),
            pltpu.VMEM((N,), jnp.float32),
            pltpu.VMEM((SUP, CH), jnp.int32),
            pltpu.VMEM((SUP, CH), jnp.int32),
            pltpu.VMEM((SUP * CH,), jnp.float32),
            pltpu.VMEM((N,), jnp.float32),
            pltpu.VMEM((CH, HHID), jnp.float32),
            pltpu.VMEM((CH, HHID), jnp.float32),
            pltpu.VMEM((CH, HHID), jnp.float32),
            pltpu.VMEM((CH, HHID), jnp.float32),
            pltpu.VMEM((16, HHID), jnp.float32),
            pltpu.VMEM_SHARED((N, HHID), jnp.float32),
        ] + [pltpu.SemaphoreType.DMA] * 8,
    )
    return fn(h_perm, aux, src2, dst2)


def _merge_body(outp_ref, denomp_ref, h_ref, b_ref, w_ref, asr_ref, adr_ref,
                h2_ref, hp2_ref, aux_ref):
    acc = jnp.concatenate([outp_ref[0], outp_ref[1]], axis=1) + h_ref[...]
    den = jnp.sum(denomp_ref[0], axis=0) + (1.0 + 1e-16)
    node = _selu(acc / den[:, None] + b_ref[...])
    h2 = jnp.dot(node, w_ref[...], preferred_element_type=jnp.float32,
                 precision=lax.Precision.HIGHEST)
    h2_ref[...] = h2
    hp2_ref[0] = h2[:, :HHID]
    hp2_ref[1] = h2[:, HHID:]
    _write_aux(aux_ref, h2, asr_ref[...], adr_ref[...])


def _merge(outp, denomp, h, b, W, att_src, att_dst):
    return pl.pallas_call(
        _merge_body,
        grid=(NBLK,),
        in_specs=[
            pl.BlockSpec((NC, ROWB, HHID), lambda i: (0, i, 0)),
            pl.BlockSpec((1, NS, ROWB), lambda i: (i, 0, 0)),
            pl.BlockSpec((ROWB, HID), lambda i: (i, 0)),
            pl.BlockSpec((1, HID), lambda i: (0, 0)),
            pl.BlockSpec((HID, HID), lambda i: (0, 0)),
            pl.BlockSpec((1, HID), lambda i: (0, 0)),
            pl.BlockSpec((1, HID), lambda i: (0, 0)),
        ],
        out_specs=[
            pl.BlockSpec((ROWB, HID), lambda i: (i, 0)),
            pl.BlockSpec((NC, ROWB, HHID), lambda i: (0, i, 0)),
            pl.BlockSpec((1, 8, ROWB), lambda i: (i, 0, 0)),
        ],
        out_shape=[
            jax.ShapeDtypeStruct((N, HID), jnp.float32),
            jax.ShapeDtypeStruct((NC, N, HHID), jnp.float32),
            jax.ShapeDtypeStruct((NBLK, 8, ROWB), jnp.float32),
        ],
    )(outp, denomp, h, b.reshape(1, HID), W,
      att_src.reshape(1, HID), att_dst.reshape(1, HID))


def _head_body(outp_ref, denomp_ref, h_ref, b_ref, batch_ref,
               fc1w_ref, fc1b_ref, fc2w_ref, fc2b_ref,
               out_ref, sums_ref, counts_ref):
    step = pl.program_id(0)

    @pl.when(step == 0)
    def _():
        sums_ref[...] = jnp.zeros_like(sums_ref)
        counts_ref[...] = jnp.zeros_like(counts_ref)

    acc = jnp.concatenate([outp_ref[0], outp_ref[1]], axis=1) + h_ref[...]
    den = jnp.sum(denomp_ref[0], axis=0) + (1.0 + 1e-16)
    node = _selu(acc / den[:, None] + b_ref[...])
    bt = batch_ref[0, 0, :]
    oh = (bt[:, None] == lax.broadcasted_iota(jnp.int32, (ROWB, NG), 1)
          ).astype(jnp.float32)
    sums_ref[...] += jnp.dot(oh.T, node, preferred_element_type=jnp.float32,
                             precision=lax.Precision.HIGHEST)
    counts_ref[...] += jnp.sum(oh, axis=0)[:, None]

    @pl.when(step == pl.num_programs(0) - 1)
    def _():
        pooled = sums_ref[...] / jnp.maximum(counts_ref[...], 1.0)
        g = _selu(pooled)
        g1 = _selu(jnp.dot(g, fc1w_ref[...], preferred_element_type=jnp.float32,
                           precision=lax.Precision.HIGHEST) + fc1b_ref[...])
        logits = jnp.dot(g1, fc2w_ref[...], preferred_element_type=jnp.float32,
                         precision=lax.Precision.HIGHEST) + fc2b_ref[...]
        col = lax.broadcasted_iota(jnp.int32, (NG, HID), 1)
        valid = col < 2
        m = jnp.max(jnp.where(valid, logits, -jnp.inf), axis=1, keepdims=True)
        s = jnp.sum(jnp.where(valid, jnp.exp(logits - m), 0.0),
                    axis=1, keepdims=True)
        out_ref[...] = logits - (m + jnp.log(s))


def _head(outp, denomp, h, b, batch3, fc1_w, fc1_b, fc2_wp, fc2_bp):
    return pl.pallas_call(
        _head_body,
        grid=(NBLK,),
        in_specs=[
            pl.BlockSpec((NC, ROWB, HHID), lambda i: (0, i, 0)),
            pl.BlockSpec((1, NS, ROWB), lambda i: (i, 0, 0)),
            pl.BlockSpec((ROWB, HID), lambda i: (i, 0)),
            pl.BlockSpec((1, HID), lambda i: (0, 0)),
            pl.BlockSpec((1, 1, ROWB), lambda i: (i, 0, 0)),
            pl.BlockSpec((HID, NHID), lambda i: (0, 0)),
            pl.BlockSpec((1, NHID), lambda i: (0, 0)),
            pl.BlockSpec((NHID, HID), lambda i: (0, 0)),
            pl.BlockSpec((1, HID), lambda i: (0, 0)),
        ],
        out_specs=pl.BlockSpec((NG, HID), lambda i: (0, 0)),
        out_shape=jax.ShapeDtypeStruct((NG, HID), jnp.float32),
        scratch_shapes=[
            pltpu.VMEM((NG, HID), jnp.float32),
            pltpu.VMEM((NG, HID), jnp.float32),
        ],
    )(outp, denomp, h, b.reshape(1, HID), batch3,
      fc1_w, fc1_b.reshape(1, NHID), fc2_wp, fc2_bp)


def kernel(x, edge_index, batch, W1, att_src1, att_dst1, b1,
           W2, att_src2, att_dst2, b2, fc1_w, fc1_b, fc2_w, fc2_b):
    src2 = edge_index[0].reshape(NS, NCHK, CH)
    dst2 = edge_index[1].reshape(NS, NCHK, CH)
    batch3 = batch.reshape(NBLK, 1, ROWB)
    fc2_wp = jnp.pad(fc2_w, ((0, 0), (0, HID - 2)))
    fc2_bp = jnp.pad(fc2_b, (0, HID - 2)).reshape(1, HID)

    h1, h1p, aux1 = _prep(x, W1, att_src1, att_dst1)
    denomp1, outp1 = _sc_edge(h1p.reshape(NC * N, HHID), aux1, src2, dst2)
    h2, h2p, aux2 = _merge(outp1, denomp1, h1, b1, W2, att_src2, att_dst2)
    denomp2, outp2 = _sc_edge(h2p.reshape(NC * N, HHID), aux2, src2, dst2)
    out = _head(outp2, denomp2, h2, b2, batch3, fc1_w, fc1_b, fc2_wp, fc2_bp)
    return out[:, :2]


# speedup vs baseline: 44.2709x; 1.1871x over previous
import dataclasses
import functools

import jax
import jax.numpy as jnp
from jax import lax
from jax.experimental import pallas as pl
from jax.experimental.pallas import tpu as pltpu
from jax.experimental.pallas import tpu_sc as plsc

N = 10000
E = 320000
F_IN = 128
HID = 128
NHID = 64
NG = 64
NC = 2
NS = 16
HHID = HID // NC
ESUB = E // NS
CH = 80
NCHK = ESUB // CH
SUP = 10
NSUP = NCHK // SUP
ROWB = 1000
NBLK = N // ROWB

_SELU_L = 1.0507009873554805
_SELU_A = 1.6732632423543772


def _selu(x):
    return _SELU_L * jnp.where(x > 0, x, _SELU_A * (jnp.exp(x) - 1.0))


def _write_aux(aux_ref, h, asr, adr):
    a_s = jnp.sum(h * asr, axis=1)
    a_d = jnp.sum(h * adr, axis=1)
    al = a_s + a_d
    base = jnp.maximum(al, 0.2 * al)
    aux_ref[0, 0, :] = a_s
    aux_ref[0, 1, :] = a_d
    aux_ref[0, 2, :] = base


def _prep_body(x_ref, w_ref, asr_ref, adr_ref, h_ref, hp_ref, aux_ref):
    h = jnp.dot(x_ref[...], w_ref[...], preferred_element_type=jnp.float32,
                precision=lax.Precision.HIGHEST)
    h_ref[...] = h
    hp_ref[0] = h[:, :HHID]
    hp_ref[1] = h[:, HHID:]
    _write_aux(aux_ref, h, asr_ref[...], adr_ref[...])


def _prep(x, W, att_src, att_dst):
    return pl.pallas_call(
        _prep_body,
        grid=(NBLK,),
        in_specs=[
            pl.BlockSpec((ROWB, HID), lambda i: (i, 0)),
            pl.BlockSpec((HID, HID), lambda i: (0, 0)),
            pl.BlockSpec((1, HID), lambda i: (0, 0)),
            pl.BlockSpec((1, HID), lambda i: (0, 0)),
        ],
        out_specs=[
            pl.BlockSpec((ROWB, HID), lambda i: (i, 0)),
            pl.BlockSpec((NC, ROWB, HHID), lambda i: (0, i, 0)),
            pl.BlockSpec((1, 8, ROWB), lambda i: (i, 0, 0)),
        ],
        out_shape=[
            jax.ShapeDtypeStruct((N, HID), jnp.float32),
            jax.ShapeDtypeStruct((NC, N, HHID), jnp.float32),
            jax.ShapeDtypeStruct((NBLK, 8, ROWB), jnp.float32),
        ],
    )(x, W, att_src.reshape(1, HID), att_dst.reshape(1, HID))


def _sc_edge_body(h_hbm, aux_hbm, src_hbm, dst_hbm,
                  denomp_hbm, outp_hbm,
                  asrc_v, adst_v, base_v,
                  src_s, dst_s, ex_s, denom_v,
                  rows_a, rows_b, rows_c, rows_d, zbuf, out_sh,
                  gsem_a, gsem_b, gsem_c, gsem_d,
                  ssem_a, ssem_b, ssem_c, ssem_d, isem):
    core = lax.axis_index("c")
    sub = lax.axis_index("s")

    for b in range(NBLK):
        for r, tab in ((0, asrc_v), (1, adst_v), (2, base_v)):
            pltpu.async_copy(aux_hbm.at[b, r], tab.at[pl.ds(b * ROWB, ROWB)],
                             gsem_a)
    for b in range(NBLK):
        for r, tab in ((0, asrc_v), (1, adst_v), (2, base_v)):
            pltpu.make_async_copy(aux_hbm.at[b, r],
                                  tab.at[pl.ds(b * ROWB, ROWB)], gsem_a).wait()

    z16 = jnp.zeros((16,), jnp.float32)

    @pl.loop(0, 16)
    def _zb(i):
        for j in range(HHID // 16):
            zbuf[i, pl.ds(j * 16, 16)] = z16

    @pl.loop(0, N // 16)
    def _zd(k):
        denom_v[pl.ds(k * 16, 16)] = z16

    @pl.loop(0, (N // 16 + NS - 1) // NS)
    def _zo(k):
        idx = k * NS + sub

        @pl.when(idx < N // 16)
        def _():
            pltpu.async_copy(zbuf, out_sh.at[pl.ds(idx * 16, 16)], ssem_a)

    @pl.loop(0, (N // 16 + NS - 1) // NS)
    def _zow(k):
        idx = k * NS + sub

        @pl.when(idx < N // 16)
        def _():
            pltpu.make_async_copy(zbuf, out_sh.at[pl.ds(idx * 16, 16)],
                                  ssem_a).wait()

    plsc.subcore_barrier()

    rr = (rows_a, rows_b, rows_c, rows_d)
    gs = (gsem_a, gsem_b, gsem_c, gsem_d)
    ss = (ssem_a, ssem_b, ssem_c, ssem_d)

    def _wait_bytes(rows, sem):
        pltpu.make_async_copy(h_hbm.at[src_s.at[0, 0]], rows, sem).wait()

    def _scale(c, p, rows):
        @plsc.parallel_loop(0, CH, unroll=8)
        def _s(i):
            cv = plsc.load_gather(ex_s, [jnp.full((16,), c * CH + i, jnp.int32)])
            for j in range(HHID // 16):
                rows[i, pl.ds(j * 16, 16)] = rows[i, pl.ds(j * 16, 16)] * cv

    def _issue_idx(S, p):
        pltpu.async_copy(src_hbm.at[sub, pl.ds(S * SUP, SUP)], src_s.at[p],
                         isem)
        pltpu.async_copy(dst_hbm.at[sub, pl.ds(S * SUP, SUP)], dst_s.at[p],
                         isem)

    def _wait_idx(p):
        pltpu.make_async_copy(src_hbm.at[sub, pl.ds(0, SUP)], src_s.at[p],
                              isem).wait()
        pltpu.make_async_copy(src_hbm.at[sub, pl.ds(0, SUP)], dst_s.at[p],
                              isem).wait()

    _issue_idx(0, 0)

    @pl.loop(0, NSUP)
    def _edges(S):
        p = lax.rem(S, 2)
        _wait_idx(p)

        @pl.when(S + 1 < NSUP)
        def _():
            _issue_idx(S + 1, 1 - p)

        @pl.loop(0, SUP)
        def _ex(c):
            for k in range(CH // 16):
                s16 = src_s[p, c, pl.ds(k * 16, 16)]
                d16 = dst_s[p, c, pl.ds(k * 16, 16)]
                a_s = plsc.load_gather(asrc_v, [s16])
                a_d = plsc.load_gather(adst_v, [d16])
                b_d = plsc.load_gather(base_v, [d16])
                al = a_s + a_d
                al = jnp.maximum(al, 0.2 * al)
                ex = jnp.exp(al - b_d)
                ex_s[pl.ds(c * CH + k * 16, 16)] = ex
                plsc.addupdate_scatter(denom_v, [d16], ex)
                src_s[p, c, pl.ds(k * 16, 16)] = s16 + core * N

        for c in range(3):
            pltpu.async_copy(h_hbm.at[src_s.at[p, c]], rr[c], gs[c])
        for c in range(SUP):
            b = c % 4
            _wait_bytes(rr[b], gs[b])
            _scale(c, p, rr[b])
            pltpu.async_copy(rr[b], out_sh.at[dst_s.at[p, c]], ss[b], add=True)
            if c + 3 < SUP:
                b3 = (c + 3) % 4
                if c - 1 >= 0:
                    _wait_bytes(rr[b3], ss[b3])
                pltpu.async_copy(h_hbm.at[src_s.at[p, c + 3]], rr[b3], gs[b3])
        for c in range(SUP - 4, SUP):
            _wait_bytes(rr[c % 4], ss[c % 4])

    @pl.when(core == 0)
    def _():
        for b in range(NBLK):
            pltpu.async_copy(denom_v.at[pl.ds(b * ROWB, ROWB)],
                             denomp_hbm.at[b, sub], gsem_a)
        for b in range(NBLK):
            pltpu.make_async_copy(denom_v.at[pl.ds(b * ROWB, ROWB)],
                                  denomp_hbm.at[b, sub], gsem_a).wait()

    plsc.subcore_barrier()

    @pl.loop(0, (N // 16 + NS - 1) // NS)
    def _wb(k):
        idx = k * NS + sub

        @pl.when(idx < N // 16)
        def _():
            pltpu.async_copy(out_sh.at[pl.ds(idx * 16, 16)],
                             outp_hbm.at[core, pl.ds(idx * 16, 16)], ssem_b)

    @pl.loop(0, (N // 16 + NS - 1) // NS)
    def _wbw(k):
        idx = k * NS + sub

        @pl.when(idx < N // 16)
        def _():
            pltpu.make_async_copy(out_sh.at[pl.ds(idx * 16, 16)],
                                  outp_hbm.at[core, pl.ds(idx * 16, 16)],
                                  ssem_b).wait()


_SC_PARAMS = pltpu.CompilerParams()
for _f, _v in (("needs_layout_passes", False), ("use_tc_tiling_on_sc", False)):
    if _f in pltpu.CompilerParams.__dataclass_fields__:
        _SC_PARAMS = dataclasses.replace(_SC_PARAMS, **{_f: _v})


def _sc_edge(h_perm, aux, src2, dst2):
    mesh = plsc.VectorSubcoreMesh(core_axis_name="c", subcore_axis_name="s")
    fn = pl.kernel(
        _sc_edge_body,
        mesh=mesh,
        compiler_params=_SC_PARAMS,
        out_type=[
            jax.ShapeDtypeStruct((NBLK, NS, ROWB), jnp.float32),
            jax.ShapeDtypeStruct((NC, N, HHID), jnp.float32),
        ],
        scratch_types=[
            pltpu.VMEM((N,), jnp.float32),
            pltpu.VMEM((N,), jnp.float32),
            pltpu.VMEM((N,), jnp.float32),
            pltpu.VMEM((2, SUP, CH), jnp.int32),
            pltpu.VMEM((2, SUP, CH), jnp.int32),
            pltpu.VMEM((SUP * CH,), jnp.float32),
            pltpu.VMEM((N,), jnp.float32),
            pltpu.VMEM((CH, HHID), jnp.float32),
            pltpu.VMEM((CH, HHID), jnp.float32),
            pltpu.VMEM((CH, HHID), jnp.float32),
            pltpu.VMEM((CH, HHID), jnp.float32),
            pltpu.VMEM((16, HHID), jnp.float32),
            pltpu.VMEM_SHARED((N, HHID), jnp.float32),
        ] + [pltpu.SemaphoreType.DMA] * 9,
    )
    return fn(h_perm, aux, src2, dst2)


def _merge_body(outp_ref, denomp_ref, h_ref, b_ref, w_ref, asr_ref, adr_ref,
                h2_ref, hp2_ref, aux_ref):
    acc = jnp.concatenate([outp_ref[0], outp_ref[1]], axis=1) + h_ref[...]
    den = jnp.sum(denomp_ref[0], axis=0) + (1.0 + 1e-16)
    node = _selu(acc / den[:, None] + b_ref[...])
    h2 = jnp.dot(node, w_ref[...], preferred_element_type=jnp.float32,
                 precision=lax.Precision.HIGHEST)
    h2_ref[...] = h2
    hp2_ref[0] = h2[:, :HHID]
    hp2_ref[1] = h2[:, HHID:]
    _write_aux(aux_ref, h2, asr_ref[...], adr_ref[...])


def _merge(outp, denomp, h, b, W, att_src, att_dst):
    return pl.pallas_call(
        _merge_body,
        grid=(NBLK,),
        in_specs=[
            pl.BlockSpec((NC, ROWB, HHID), lambda i: (0, i, 0)),
            pl.BlockSpec((1, NS, ROWB), lambda i: (i, 0, 0)),
            pl.BlockSpec((ROWB, HID), lambda i: (i, 0)),
            pl.BlockSpec((1, HID), lambda i: (0, 0)),
            pl.BlockSpec((HID, HID), lambda i: (0, 0)),
            pl.BlockSpec((1, HID), lambda i: (0, 0)),
            pl.BlockSpec((1, HID), lambda i: (0, 0)),
        ],
        out_specs=[
            pl.BlockSpec((ROWB, HID), lambda i: (i, 0)),
            pl.BlockSpec((NC, ROWB, HHID), lambda i: (0, i, 0)),
            pl.BlockSpec((1, 8, ROWB), lambda i: (i, 0, 0)),
        ],
        out_shape=[
            jax.ShapeDtypeStruct((N, HID), jnp.float32),
            jax.ShapeDtypeStruct((NC, N, HHID), jnp.float32),
            jax.ShapeDtypeStruct((NBLK, 8, ROWB), jnp.float32),
        ],
    )(outp, denomp, h, b.reshape(1, HID), W,
      att_src.reshape(1, HID), att_dst.reshape(1, HID))


def _head_body(outp_ref, denomp_ref, h_ref, b_ref, batch_ref,
               fc1w_ref, fc1b_ref, fc2w_ref, fc2b_ref,
               out_ref, sums_ref, counts_ref):
    step = pl.program_id(0)

    @pl.when(step == 0)
    def _():
        sums_ref[...] = jnp.zeros_like(sums_ref)
        counts_ref[...] = jnp.zeros_like(counts_ref)

    acc = jnp.concatenate([outp_ref[0], outp_ref[1]], axis=1) + h_ref[...]
    den = jnp.sum(denomp_ref[0], axis=0) + (1.0 + 1e-16)
    node = _selu(acc / den[:, None] + b_ref[...])
    bt = batch_ref[0, 0, :]
    oh = (bt[:, None] == lax.broadcasted_iota(jnp.int32, (ROWB, NG), 1)
          ).astype(jnp.float32)
    sums_ref[...] += jnp.dot(oh.T, node, preferred_element_type=jnp.float32,
                             precision=lax.Precision.HIGHEST)
    counts_ref[...] += jnp.sum(oh, axis=0)[:, None]

    @pl.when(step == pl.num_programs(0) - 1)
    def _():
        pooled = sums_ref[...] / jnp.maximum(counts_ref[...], 1.0)
        g = _selu(pooled)
        g1 = _selu(jnp.dot(g, fc1w_ref[...], preferred_element_type=jnp.float32,
                           precision=lax.Precision.HIGHEST) + fc1b_ref[...])
        logits = jnp.dot(g1, fc2w_ref[...], preferred_element_type=jnp.float32,
                         precision=lax.Precision.HIGHEST) + fc2b_ref[...]
        col = lax.broadcasted_iota(jnp.int32, (NG, HID), 1)
        valid = col < 2
        m = jnp.max(jnp.where(valid, logits, -jnp.inf), axis=1, keepdims=True)
        s = jnp.sum(jnp.where(valid, jnp.exp(logits - m), 0.0),
                    axis=1, keepdims=True)
        out_ref[...] = logits - (m + jnp.log(s))


def _head(outp, denomp, h, b, batch3, fc1_w, fc1_b, fc2_wp, fc2_bp):
    return pl.pallas_call(
        _head_body,
        grid=(NBLK,),
        in_specs=[
            pl.BlockSpec((NC, ROWB, HHID), lambda i: (0, i, 0)),
            pl.BlockSpec((1, NS, ROWB), lambda i: (i, 0, 0)),
            pl.BlockSpec((ROWB, HID), lambda i: (i, 0)),
            pl.BlockSpec((1, HID), lambda i: (0, 0)),
            pl.BlockSpec((1, 1, ROWB), lambda i: (i, 0, 0)),
            pl.BlockSpec((HID, NHID), lambda i: (0, 0)),
            pl.BlockSpec((1, NHID), lambda i: (0, 0)),
            pl.BlockSpec((NHID, HID), lambda i: (0, 0)),
            pl.BlockSpec((1, HID), lambda i: (0, 0)),
        ],
        out_specs=pl.BlockSpec((NG, HID), lambda i: (0, 0)),
        out_shape=jax.ShapeDtypeStruct((NG, HID), jnp.float32),
        scratch_shapes=[
            pltpu.VMEM((NG, HID), jnp.float32),
            pltpu.VMEM((NG, HID), jnp.float32),
        ],
    )(outp, denomp, h, b.reshape(1, HID), batch3,
      fc1_w, fc1_b.reshape(1, NHID), fc2_wp, fc2_bp)


def kernel(x, edge_index, batch, W1, att_src1, att_dst1, b1,
           W2, att_src2, att_dst2, b2, fc1_w, fc1_b, fc2_w, fc2_b):
    src2 = edge_index[0].reshape(NS, NCHK, CH)
    dst2 = edge_index[1].reshape(NS, NCHK, CH)
    batch3 = batch.reshape(NBLK, 1, ROWB)
    fc2_wp = jnp.pad(fc2_w, ((0, 0), (0, HID - 2)))
    fc2_bp = jnp.pad(fc2_b, (0, HID - 2)).reshape(1, HID)

    h1, h1p, aux1 = _prep(x, W1, att_src1, att_dst1)
    denomp1, outp1 = _sc_edge(h1p.reshape(NC * N, HHID), aux1, src2, dst2)
    h2, h2p, aux2 = _merge(outp1, denomp1, h1, b1, W2, att_src2, att_dst2)
    denomp2, outp2 = _sc_edge(h2p.reshape(NC * N, HHID), aux2, src2, dst2)
    out = _head(outp2, denomp2, h2, b2, batch3, fc1_w, fc1_b, fc2_wp, fc2_bp)
    return out[:, :2]
